# baseline (speedup 1.0000x reference)
import functools

import jax
import jax.numpy as jnp
import numpy as np
from jax.experimental import pallas as pl
from jax.experimental.pallas import tpu as pltpu


def _dft_mats(C: int, FP: int):
    F = C // 2 + 1
    c = np.arange(C)[:, None].astype(np.float64)
    f = np.arange(FP)[None, :].astype(np.float64)
    ang = 2.0 * np.pi * c * f / C
    valid = (f < F).astype(np.float64)
    cosm = np.cos(ang) * valid
    sinm = -np.sin(ang) * valid
    fwd = np.concatenate([cosm, sinm], axis=1)
    w = np.full((FP, 1), 2.0 / C)
    w[0, 0] = 1.0 / C
    if F - 1 < FP:
        w[F - 1, 0] = 1.0 / C
    angT = 2.0 * np.pi * np.arange(FP)[:, None].astype(np.float64) * np.arange(C)[None, :] / C
    validT = (np.arange(FP)[:, None] < F).astype(np.float64)
    icos = np.cos(angT) * w * validT
    isin = -np.sin(angT) * w * validT
    inv = np.concatenate([icos, isin], axis=0)
    return fwd.astype(np.float32), inv.astype(np.float32)


def _fan_block(x_ref, fwd_ref, inv_ref, w1_ref, b1_ref, w2_ref, b2_ref,
               w3_ref, b3_ref, o_ref, *, F: int, FP: int, K: int):
    x = x_ref[...]
    hi = jax.lax.Precision.HIGHEST
    spec = jnp.dot(x, fwd_ref[...], preferred_element_type=jnp.float32,
                   precision=hi)
    re = spec[:, :FP]
    im = spec[:, FP:]
    mag = jnp.sqrt(re * re + im * im)
    cols = jax.lax.broadcasted_iota(jnp.int32, mag.shape, 1)
    neg_inf = jnp.float32(-jnp.inf)
    mag = jnp.where(cols < F, mag, neg_inf)

    def body(_, carry):
        m, keep = carry
        mx = jnp.max(m, axis=1, keepdims=True)
        sel = jnp.min(jnp.where(m == mx, cols, jnp.int32(1 << 20)),
                      axis=1, keepdims=True)
        newly = cols == sel
        keep = jnp.where(newly, jnp.float32(1.0), keep)
        m = jnp.where(newly, neg_inf, m)
        return m, keep

    _, keep = jax.lax.fori_loop(
        0, K, body, (mag, jnp.zeros(mag.shape, jnp.float32)))

    spec_m = spec * jnp.concatenate([keep, keep], axis=1)
    x_filt = jnp.dot(spec_m, inv_ref[...],
                     preferred_element_type=jnp.float32, precision=hi)
    pf = jnp.maximum(
        jnp.dot(x_filt, w1_ref[...], preferred_element_type=jnp.float32,
                precision=hi) + b1_ref[...], 0.0)
    comb = jnp.concatenate([pf, x], axis=1)
    h = jnp.maximum(
        jnp.dot(comb, w2_ref[...], preferred_element_type=jnp.float32,
                precision=hi) + b2_ref[...], 0.0)
    out_mlp = jnp.dot(h, w3_ref[...], preferred_element_type=jnp.float32,
                      precision=hi) + b3_ref[...]
    o_ref[...] = (x - x_filt) + out_mlp


@jax.jit
def kernel(x, W1, b1, W2, b2, W3, b3):
    B, S, C = x.shape
    F = C // 2 + 1
    FP = ((F + 127) // 128) * 128
    K = min(20, F)
    T = B * S
    TB = 256 if T % 256 == 0 else T

    fwd_np, inv_np = _dft_mats(C, FP)
    fwd = jnp.asarray(fwd_np)
    inv = jnp.asarray(inv_np)

    H1 = W1.shape[1]
    H1P = 128
    w1p = jnp.zeros((C, H1P), jnp.float32).at[:, :H1].set(W1)
    b1p = jnp.zeros((1, H1P), jnp.float32).at[0, :H1].set(b1)
    H2 = W2.shape[1]
    w2p = jnp.zeros((H1P + C, H2), jnp.float32)
    w2p = w2p.at[:H1, :].set(W2[:H1, :]).at[H1P:, :].set(W2[H1:, :])
    b2r = b2.reshape(1, H2)
    b3r = b3.reshape(1, C)

    xt = x.reshape(T, C)
    full = lambda shape: pl.BlockSpec(shape, lambda i: (0, 0))
    out = pl.pallas_call(
        functools.partial(_fan_block, F=F, FP=FP, K=K),
        grid=(T // TB,),
        in_specs=[
            pl.BlockSpec((TB, C), lambda i: (i, 0)),
            full((C, 2 * FP)),
            full((2 * FP, C)),
            full((C, H1P)),
            full((1, H1P)),
            full((H1P + C, H2)),
            full((1, H2)),
            full((H2, C)),
            full((1, C)),
        ],
        out_specs=pl.BlockSpec((TB, C), lambda i: (i, 0)),
        out_shape=jax.ShapeDtypeStruct((T, C), jnp.float32),
        compiler_params=pltpu.CompilerParams(
            dimension_semantics=("parallel",)),
    )(xt, fwd, inv, w1p, b1p, w2p, b2r, W3, b3r)
    return out.reshape(B, S, C)


# speedup vs baseline: 1.2678x; 1.2678x over previous
import functools

import jax
import jax.numpy as jnp
import numpy as np
from jax.experimental import pallas as pl
from jax.experimental.pallas import tpu as pltpu


def _dft_mats(C: int, FP: int):
    F = C // 2 + 1
    c = np.arange(C)[:, None].astype(np.float64)
    f = np.arange(FP)[None, :].astype(np.float64)
    ang = 2.0 * np.pi * c * f / C
    valid = (f < F).astype(np.float64)
    cosm = np.cos(ang) * valid
    sinm = -np.sin(ang) * valid
    fwd = np.concatenate([cosm, sinm], axis=1)
    w = np.full((FP, 1), 2.0 / C)
    w[0, 0] = 1.0 / C
    if F - 1 < FP:
        w[F - 1, 0] = 1.0 / C
    angT = 2.0 * np.pi * np.arange(FP)[:, None].astype(np.float64) * np.arange(C)[None, :] / C
    validT = (np.arange(FP)[:, None] < F).astype(np.float64)
    icos = np.cos(angT) * w * validT
    isin = -np.sin(angT) * w * validT
    inv = np.concatenate([icos, isin], axis=0)
    return fwd.astype(np.float32), inv.astype(np.float32)


def _fan_block(x_ref, fwd_ref, inv_ref, w1_ref, b1_ref, w2_ref, b2_ref,
               w3_ref, b3_ref, o_ref, *, F: int, FP: int, K: int):
    x = x_ref[...]
    hi = jax.lax.Precision.HIGHEST
    spec = jnp.dot(x, fwd_ref[...], preferred_element_type=jnp.float32,
                   precision=hi)
    re = spec[:, :FP]
    im = spec[:, FP:]
    mag = jnp.sqrt(re * re + im * im)
    cols = jax.lax.broadcasted_iota(jnp.int32, mag.shape, 1)
    neg_inf = jnp.float32(-jnp.inf)
    mag = jnp.where(cols < F, mag, neg_inf)

    def body(_, carry):
        m, keep = carry
        mx = jnp.max(m, axis=1, keepdims=True)
        sel = jnp.min(jnp.where(m == mx, cols, jnp.int32(1 << 20)),
                      axis=1, keepdims=True)
        newly = cols == sel
        keep = jnp.where(newly, jnp.float32(1.0), keep)
        m = jnp.where(newly, neg_inf, m)
        return m, keep

    _, keep = jax.lax.fori_loop(
        0, K, body, (mag, jnp.zeros(mag.shape, jnp.float32)))

    lo = jax.lax.Precision.DEFAULT
    spec_m = spec * jnp.concatenate([keep, keep], axis=1)
    x_filt = jnp.dot(spec_m, inv_ref[...],
                     preferred_element_type=jnp.float32, precision=lo)
    pf = jnp.maximum(
        jnp.dot(x_filt, w1_ref[...], preferred_element_type=jnp.float32,
                precision=lo) + b1_ref[...], 0.0)
    comb = jnp.concatenate([pf, x], axis=1)
    h = jnp.maximum(
        jnp.dot(comb, w2_ref[...], preferred_element_type=jnp.float32,
                precision=lo) + b2_ref[...], 0.0)
    out_mlp = jnp.dot(h, w3_ref[...], preferred_element_type=jnp.float32,
                      precision=lo) + b3_ref[...]
    o_ref[...] = (x - x_filt) + out_mlp


@jax.jit
def kernel(x, W1, b1, W2, b2, W3, b3):
    B, S, C = x.shape
    F = C // 2 + 1
    FP = ((F + 127) // 128) * 128
    K = min(20, F)
    T = B * S
    TB = 256 if T % 256 == 0 else T

    fwd_np, inv_np = _dft_mats(C, FP)
    fwd = jnp.asarray(fwd_np)
    inv = jnp.asarray(inv_np)

    H1 = W1.shape[1]
    H1P = 128
    w1p = jnp.zeros((C, H1P), jnp.float32).at[:, :H1].set(W1)
    b1p = jnp.zeros((1, H1P), jnp.float32).at[0, :H1].set(b1)
    H2 = W2.shape[1]
    w2p = jnp.zeros((H1P + C, H2), jnp.float32)
    w2p = w2p.at[:H1, :].set(W2[:H1, :]).at[H1P:, :].set(W2[H1:, :])
    b2r = b2.reshape(1, H2)
    b3r = b3.reshape(1, C)

    xt = x.reshape(T, C)
    full = lambda shape: pl.BlockSpec(shape, lambda i: (0, 0))
    out = pl.pallas_call(
        functools.partial(_fan_block, F=F, FP=FP, K=K),
        grid=(T // TB,),
        in_specs=[
            pl.BlockSpec((TB, C), lambda i: (i, 0)),
            full((C, 2 * FP)),
            full((2 * FP, C)),
            full((C, H1P)),
            full((1, H1P)),
            full((H1P + C, H2)),
            full((1, H2)),
            full((H2, C)),
            full((1, C)),
        ],
        out_specs=pl.BlockSpec((TB, C), lambda i: (i, 0)),
        out_shape=jax.ShapeDtypeStruct((T, C), jnp.float32),
        compiler_params=pltpu.CompilerParams(
            dimension_semantics=("parallel",)),
    )(xt, fwd, inv, w1p, b1p, w2p, b2r, W3, b3r)
    return out.reshape(B, S, C)


# speedup vs baseline: 1.6365x; 1.2908x over previous
import functools

import jax
import jax.numpy as jnp
import numpy as np
from jax.experimental import pallas as pl
from jax.experimental.pallas import tpu as pltpu


def _dft_mats(C: int, FP: int):
    F = C // 2 + 1
    c = np.arange(C)[:, None].astype(np.float64)
    f = np.arange(FP)[None, :].astype(np.float64)
    ang = 2.0 * np.pi * c * f / C
    valid = (f < F).astype(np.float64)
    cosm = np.cos(ang) * valid
    sinm = -np.sin(ang) * valid
    fwd = np.concatenate([cosm, sinm], axis=1)
    w = np.full((FP, 1), 2.0 / C)
    w[0, 0] = 1.0 / C
    if F - 1 < FP:
        w[F - 1, 0] = 1.0 / C
    angT = 2.0 * np.pi * np.arange(FP)[:, None].astype(np.float64) * np.arange(C)[None, :] / C
    validT = (np.arange(FP)[:, None] < F).astype(np.float64)
    icos = np.cos(angT) * w * validT
    isin = -np.sin(angT) * w * validT
    inv = np.concatenate([icos, isin], axis=0)
    return fwd.astype(np.float32), inv.astype(np.float32)


def _fan_block(x_ref, fwd_ref, inv_ref, w1_ref, b1_ref, w2_ref, b2_ref,
               w3_ref, b3_ref, o_ref, *, F: int, FP: int, K: int):
    x = x_ref[...]
    hi = jax.lax.Precision.HIGHEST
    spec = jnp.dot(x, fwd_ref[...], preferred_element_type=jnp.float32,
                   precision=hi)
    re = spec[:, :FP]
    im = spec[:, FP:]
    mag = jnp.sqrt(re * re + im * im)
    cols = jax.lax.broadcasted_iota(jnp.int32, mag.shape, 1)
    neg_inf = jnp.float32(-jnp.inf)
    mag = jnp.where(cols < F, mag, neg_inf)

    def body(_, carry):
        m, keep = carry
        mx = jnp.max(m, axis=1, keepdims=True)
        sel = jnp.min(jnp.where(m == mx, cols, jnp.int32(1 << 20)),
                      axis=1, keepdims=True)
        newly = cols == sel
        keep = jnp.where(newly, jnp.float32(1.0), keep)
        m = jnp.where(newly, neg_inf, m)
        return m, keep

    _, keep = jax.lax.fori_loop(
        0, K, body, (mag, jnp.zeros(mag.shape, jnp.float32)))

    lo = jax.lax.Precision.DEFAULT
    spec_m = spec * jnp.concatenate([keep, keep], axis=1)
    x_filt = jnp.dot(spec_m, inv_ref[...],
                     preferred_element_type=jnp.float32, precision=lo)
    pf = jnp.maximum(
        jnp.dot(x_filt, w1_ref[...], preferred_element_type=jnp.float32,
                precision=lo) + b1_ref[...], 0.0)
    comb = jnp.concatenate([pf, x], axis=1)
    h = jnp.maximum(
        jnp.dot(comb, w2_ref[...], preferred_element_type=jnp.float32,
                precision=lo) + b2_ref[...], 0.0)
    out_mlp = jnp.dot(h, w3_ref[...], preferred_element_type=jnp.float32,
                      precision=lo) + b3_ref[...]
    o_ref[...] = (x - x_filt) + out_mlp


@jax.jit
def kernel(x, W1, b1, W2, b2, W3, b3):
    B, S, C = x.shape
    F = C // 2 + 1
    FP = ((F + 127) // 128) * 128
    K = min(20, F)
    T = B * S
    TB = 512 if T % 512 == 0 else T

    fwd_np, inv_np = _dft_mats(C, FP)
    fwd = jnp.asarray(fwd_np)
    inv = jnp.asarray(inv_np)

    H1 = W1.shape[1]
    H1P = 128
    w1p = jnp.zeros((C, H1P), jnp.float32).at[:, :H1].set(W1)
    b1p = jnp.zeros((1, H1P), jnp.float32).at[0, :H1].set(b1)
    H2 = W2.shape[1]
    w2p = jnp.zeros((H1P + C, H2), jnp.float32)
    w2p = w2p.at[:H1, :].set(W2[:H1, :]).at[H1P:, :].set(W2[H1:, :])
    b2r = b2.reshape(1, H2)
    b3r = b3.reshape(1, C)

    xt = x.reshape(T, C)
    full = lambda shape: pl.BlockSpec(shape, lambda i: (0, 0))
    out = pl.pallas_call(
        functools.partial(_fan_block, F=F, FP=FP, K=K),
        grid=(T // TB,),
        in_specs=[
            pl.BlockSpec((TB, C), lambda i: (i, 0)),
            full((C, 2 * FP)),
            full((2 * FP, C)),
            full((C, H1P)),
            full((1, H1P)),
            full((H1P + C, H2)),
            full((1, H2)),
            full((H2, C)),
            full((1, C)),
        ],
        out_specs=pl.BlockSpec((TB, C), lambda i: (i, 0)),
        out_shape=jax.ShapeDtypeStruct((T, C), jnp.float32),
        compiler_params=pltpu.CompilerParams(
            dimension_semantics=("parallel",)),
    )(xt, fwd, inv, w1p, b1p, w2p, b2r, W3, b3r)
    return out.reshape(B, S, C)


# speedup vs baseline: 1.6806x; 1.0269x over previous
import functools

import jax
import jax.numpy as jnp
import numpy as np
from jax.experimental import pallas as pl
from jax.experimental.pallas import tpu as pltpu


def _dft_mats(C: int, FP: int):
    F = C // 2 + 1
    c = np.arange(C)[:, None].astype(np.float64)
    f = np.arange(FP)[None, :].astype(np.float64)
    ang = 2.0 * np.pi * c * f / C
    valid = (f < F).astype(np.float64)
    cosm = np.cos(ang) * valid
    sinm = -np.sin(ang) * valid
    fwd = np.concatenate([cosm, sinm], axis=1)
    w = np.full((FP, 1), 2.0 / C)
    w[0, 0] = 1.0 / C
    if F - 1 < FP:
        w[F - 1, 0] = 1.0 / C
    angT = 2.0 * np.pi * np.arange(FP)[:, None].astype(np.float64) * np.arange(C)[None, :] / C
    validT = (np.arange(FP)[:, None] < F).astype(np.float64)
    icos = np.cos(angT) * w * validT
    isin = -np.sin(angT) * w * validT
    inv = np.concatenate([icos, isin], axis=0)
    return fwd.astype(np.float32), inv.astype(np.float32)


def _fan_block(x_ref, fwd_ref, inv_ref, w1_ref, b1_ref, w2_ref, b2_ref,
               w3_ref, b3_ref, o_ref, *, F: int, FP: int, K: int):
    x = x_ref[...]
    hi = jax.lax.Precision.HIGHEST
    spec = jnp.dot(x, fwd_ref[...], preferred_element_type=jnp.float32,
                   precision=hi)
    re = spec[:, :FP]
    im = spec[:, FP:]
    mag = jnp.sqrt(re * re + im * im)
    cols = jax.lax.broadcasted_iota(jnp.int32, mag.shape, 1)
    neg_inf = jnp.float32(-jnp.inf)
    mag = jnp.where(cols < F, mag, neg_inf)

    def body(_, carry):
        m, keep = carry
        mx = jnp.max(m, axis=1, keepdims=True)
        sel = jnp.min(jnp.where(m == mx, cols, jnp.int32(1 << 20)),
                      axis=1, keepdims=True)
        newly = cols == sel
        keep = jnp.where(newly, jnp.float32(1.0), keep)
        m = jnp.where(newly, neg_inf, m)
        return m, keep

    _, keep = jax.lax.fori_loop(
        0, K, body, (mag, jnp.zeros(mag.shape, jnp.float32)))

    lo = jax.lax.Precision.DEFAULT
    spec_m = spec * jnp.concatenate([keep, keep], axis=1)
    x_filt = jnp.dot(spec_m, inv_ref[...],
                     preferred_element_type=jnp.float32, precision=lo)
    pf = jnp.maximum(
        jnp.dot(x_filt, w1_ref[...], preferred_element_type=jnp.float32,
                precision=lo) + b1_ref[...], 0.0)
    comb = jnp.concatenate([pf, x], axis=1)
    h = jnp.maximum(
        jnp.dot(comb, w2_ref[...], preferred_element_type=jnp.float32,
                precision=lo) + b2_ref[...], 0.0)
    out_mlp = jnp.dot(h, w3_ref[...], preferred_element_type=jnp.float32,
                      precision=lo) + b3_ref[...]
    o_ref[...] = (x - x_filt) + out_mlp


@jax.jit
def kernel(x, W1, b1, W2, b2, W3, b3):
    B, S, C = x.shape
    F = C // 2 + 1
    FP = ((F + 127) // 128) * 128
    K = min(20, F)
    T = B * S
    TB = 1024 if T % 1024 == 0 else T

    fwd_np, inv_np = _dft_mats(C, FP)
    fwd = jnp.asarray(fwd_np)
    inv = jnp.asarray(inv_np)

    H1 = W1.shape[1]
    H1P = 128
    w1p = jnp.zeros((C, H1P), jnp.float32).at[:, :H1].set(W1)
    b1p = jnp.zeros((1, H1P), jnp.float32).at[0, :H1].set(b1)
    H2 = W2.shape[1]
    w2p = jnp.zeros((H1P + C, H2), jnp.float32)
    w2p = w2p.at[:H1, :].set(W2[:H1, :]).at[H1P:, :].set(W2[H1:, :])
    b2r = b2.reshape(1, H2)
    b3r = b3.reshape(1, C)

    xt = x.reshape(T, C)
    full = lambda shape: pl.BlockSpec(shape, lambda i: (0, 0))
    out = pl.pallas_call(
        functools.partial(_fan_block, F=F, FP=FP, K=K),
        grid=(T // TB,),
        in_specs=[
            pl.BlockSpec((TB, C), lambda i: (i, 0)),
            full((C, 2 * FP)),
            full((2 * FP, C)),
            full((C, H1P)),
            full((1, H1P)),
            full((H1P + C, H2)),
            full((1, H2)),
            full((H2, C)),
            full((1, C)),
        ],
        out_specs=pl.BlockSpec((TB, C), lambda i: (i, 0)),
        out_shape=jax.ShapeDtypeStruct((T, C), jnp.float32),
        compiler_params=pltpu.CompilerParams(
            dimension_semantics=("parallel",)),
    )(xt, fwd, inv, w1p, b1p, w2p, b2r, W3, b3r)
    return out.reshape(B, S, C)


# speedup vs baseline: 2.0920x; 1.2448x over previous
import functools

import jax
import jax.numpy as jnp
import numpy as np
from jax.experimental import pallas as pl
from jax.experimental.pallas import tpu as pltpu


def _dft_mats(C: int, FP: int):
    F = C // 2 + 1
    c = np.arange(C)[:, None].astype(np.float64)
    f = np.arange(FP)[None, :].astype(np.float64)
    ang = 2.0 * np.pi * c * f / C
    valid = (f < F).astype(np.float64)
    cosm = np.cos(ang) * valid
    sinm = -np.sin(ang) * valid
    fwd = np.concatenate([cosm, sinm], axis=1)
    w = np.full((FP, 1), 2.0 / C)
    w[0, 0] = 1.0 / C
    if F - 1 < FP:
        w[F - 1, 0] = 1.0 / C
    angT = 2.0 * np.pi * np.arange(FP)[:, None].astype(np.float64) * np.arange(C)[None, :] / C
    validT = (np.arange(FP)[:, None] < F).astype(np.float64)
    icos = np.cos(angT) * w * validT
    isin = -np.sin(angT) * w * validT
    inv = np.concatenate([icos, isin], axis=0)
    return fwd.astype(np.float32), inv.astype(np.float32)


def _fan_block(x_ref, fwd_ref, inv_ref, w1_ref, b1_ref, w2_ref, b2_ref,
               w3_ref, b3_ref, o_ref, *, F: int, FP: int, K: int):
    x = x_ref[...]
    hi = jax.lax.Precision.HIGHEST
    spec = jnp.dot(x, fwd_ref[...], preferred_element_type=jnp.float32,
                   precision=hi)
    re = spec[:, :FP]
    im = spec[:, FP:]
    mag = re * re + im * im
    cols = jax.lax.broadcasted_iota(jnp.int32, mag.shape, 1)
    neg_inf = jnp.float32(-jnp.inf)
    mag = jnp.where(cols < F, mag, neg_inf)

    def body(_, carry):
        m, keep = carry
        mx = jnp.max(m, axis=1, keepdims=True)
        sel = jnp.min(jnp.where(m == mx, cols, jnp.int32(1 << 20)),
                      axis=1, keepdims=True)
        newly = cols == sel
        keep = jnp.where(newly, jnp.float32(1.0), keep)
        m = jnp.where(newly, neg_inf, m)
        return m, keep

    _, keep = jax.lax.fori_loop(
        0, K, body, (mag, jnp.zeros(mag.shape, jnp.float32)))

    lo = jax.lax.Precision.DEFAULT
    spec_m = spec * jnp.concatenate([keep, keep], axis=1)
    x_filt = jnp.dot(spec_m, inv_ref[...],
                     preferred_element_type=jnp.float32, precision=lo)
    pf = jnp.maximum(
        jnp.dot(x_filt, w1_ref[...], preferred_element_type=jnp.float32,
                precision=lo) + b1_ref[...], 0.0)
    comb = jnp.concatenate([pf, x], axis=1)
    h = jnp.maximum(
        jnp.dot(comb, w2_ref[...], preferred_element_type=jnp.float32,
                precision=lo) + b2_ref[...], 0.0)
    out_mlp = jnp.dot(h, w3_ref[...], preferred_element_type=jnp.float32,
                      precision=lo) + b3_ref[...]
    o_ref[...] = (x - x_filt) + out_mlp


@jax.jit
def kernel(x, W1, b1, W2, b2, W3, b3):
    B, S, C = x.shape
    F = C // 2 + 1
    FP = ((F + 127) // 128) * 128
    K = min(20, F)
    T = B * S
    TB = 1024 if T % 1024 == 0 else T

    fwd_np, inv_np = _dft_mats(C, FP)
    fwd = jnp.asarray(fwd_np)
    inv = jnp.asarray(inv_np)

    H1 = W1.shape[1]
    H1P = 128
    w1p = jnp.zeros((C, H1P), jnp.float32).at[:, :H1].set(W1)
    b1p = jnp.zeros((1, H1P), jnp.float32).at[0, :H1].set(b1)
    H2 = W2.shape[1]
    w2p = jnp.zeros((H1P + C, H2), jnp.float32)
    w2p = w2p.at[:H1, :].set(W2[:H1, :]).at[H1P:, :].set(W2[H1:, :])
    b2r = b2.reshape(1, H2)
    b3r = b3.reshape(1, C)

    xt = x.reshape(T, C)
    full = lambda shape: pl.BlockSpec(shape, lambda i: (0, 0))
    out = pl.pallas_call(
        functools.partial(_fan_block, F=F, FP=FP, K=K),
        grid=(T // TB,),
        in_specs=[
            pl.BlockSpec((TB, C), lambda i: (i, 0)),
            full((C, 2 * FP)),
            full((2 * FP, C)),
            full((C, H1P)),
            full((1, H1P)),
            full((H1P + C, H2)),
            full((1, H2)),
            full((H2, C)),
            full((1, C)),
        ],
        out_specs=pl.BlockSpec((TB, C), lambda i: (i, 0)),
        out_shape=jax.ShapeDtypeStruct((T, C), jnp.float32),
        compiler_params=pltpu.CompilerParams(
            dimension_semantics=("parallel",)),
    )(xt, fwd, inv, w1p, b1p, w2p, b2r, W3, b3r)
    return out.reshape(B, S, C)


# speedup vs baseline: 4.3108x; 2.0606x over previous
import functools

import jax
import jax.numpy as jnp
import numpy as np
from jax.experimental import pallas as pl
from jax.experimental.pallas import tpu as pltpu


def _dft_mats(C: int, FP: int):
    F = C // 2 + 1
    c = np.arange(C)[:, None].astype(np.float64)
    f = np.arange(FP)[None, :].astype(np.float64)
    ang = 2.0 * np.pi * c * f / C
    valid = (f < F).astype(np.float64)
    cosm = np.cos(ang) * valid
    sinm = -np.sin(ang) * valid
    fwd = np.concatenate([cosm, sinm], axis=1)
    w = np.full((FP, 1), 2.0 / C)
    w[0, 0] = 1.0 / C
    if F - 1 < FP:
        w[F - 1, 0] = 1.0 / C
    angT = 2.0 * np.pi * np.arange(FP)[:, None].astype(np.float64) * np.arange(C)[None, :] / C
    validT = (np.arange(FP)[:, None] < F).astype(np.float64)
    icos = np.cos(angT) * w * validT
    isin = -np.sin(angT) * w * validT
    inv = np.concatenate([icos, isin], axis=0)
    return fwd.astype(np.float32), inv.astype(np.float32)


def _fan_block(x_ref, fwd_ref, inv_ref, w1_ref, b1_ref, w2_ref, b2_ref,
               w3_ref, b3_ref, o_ref, *, F: int, FP: int, K: int):
    x = x_ref[...]
    hi = jax.lax.Precision.DEFAULT
    spec = jnp.dot(x, fwd_ref[...], preferred_element_type=jnp.float32,
                   precision=hi)
    re = spec[:, :FP]
    im = spec[:, FP:]
    mag = re * re + im * im
    cols = jax.lax.broadcasted_iota(jnp.int32, mag.shape, 1)
    neg_inf = jnp.float32(-jnp.inf)
    mag = jnp.where(cols < F, mag, neg_inf)

    def body(_, m):
        mx = jnp.max(m, axis=1, keepdims=True)
        return jnp.where(m == mx, neg_inf, m)

    m_fin = jax.lax.fori_loop(0, K, body, mag)
    keep = jnp.where((m_fin == neg_inf) & (cols < F),
                     jnp.float32(1.0), jnp.float32(0.0))

    lo = jax.lax.Precision.DEFAULT
    spec_m = spec * jnp.concatenate([keep, keep], axis=1)
    x_filt = jnp.dot(spec_m, inv_ref[...],
                     preferred_element_type=jnp.float32, precision=lo)
    pf = jnp.maximum(
        jnp.dot(x_filt, w1_ref[...], preferred_element_type=jnp.float32,
                precision=lo) + b1_ref[...], 0.0)
    comb = jnp.concatenate([pf, x], axis=1)
    h = jnp.maximum(
        jnp.dot(comb, w2_ref[...], preferred_element_type=jnp.float32,
                precision=lo) + b2_ref[...], 0.0)
    out_mlp = jnp.dot(h, w3_ref[...], preferred_element_type=jnp.float32,
                      precision=lo) + b3_ref[...]
    o_ref[...] = (x - x_filt) + out_mlp


@jax.jit
def kernel(x, W1, b1, W2, b2, W3, b3):
    B, S, C = x.shape
    F = C // 2 + 1
    FP = ((F + 127) // 128) * 128
    K = min(20, F)
    T = B * S
    TB = 1024 if T % 1024 == 0 else T

    fwd_np, inv_np = _dft_mats(C, FP)
    fwd = jnp.asarray(fwd_np)
    inv = jnp.asarray(inv_np)

    H1 = W1.shape[1]
    H1P = 128
    w1p = jnp.zeros((C, H1P), jnp.float32).at[:, :H1].set(W1)
    b1p = jnp.zeros((1, H1P), jnp.float32).at[0, :H1].set(b1)
    H2 = W2.shape[1]
    w2p = jnp.zeros((H1P + C, H2), jnp.float32)
    w2p = w2p.at[:H1, :].set(W2[:H1, :]).at[H1P:, :].set(W2[H1:, :])
    b2r = b2.reshape(1, H2)
    b3r = b3.reshape(1, C)

    xt = x.reshape(T, C)
    full = lambda shape: pl.BlockSpec(shape, lambda i: (0, 0))
    out = pl.pallas_call(
        functools.partial(_fan_block, F=F, FP=FP, K=K),
        grid=(T // TB,),
        in_specs=[
            pl.BlockSpec((TB, C), lambda i: (i, 0)),
            full((C, 2 * FP)),
            full((2 * FP, C)),
            full((C, H1P)),
            full((1, H1P)),
            full((H1P + C, H2)),
            full((1, H2)),
            full((H2, C)),
            full((1, C)),
        ],
        out_specs=pl.BlockSpec((TB, C), lambda i: (i, 0)),
        out_shape=jax.ShapeDtypeStruct((T, C), jnp.float32),
        compiler_params=pltpu.CompilerParams(
            dimension_semantics=("parallel",)),
    )(xt, fwd, inv, w1p, b1p, w2p, b2r, W3, b3r)
    return out.reshape(B, S, C)
